# baseline (speedup 1.0000x reference)
import functools

import jax
import jax.numpy as jnp
from jax.experimental import pallas as pl

_K = 64
_D = 32
_BLK = 2


def _fill_body(lat_ref, c0_ref, c1_ref, c2_ref, out_ref):
    v = (3.0 * lat_ref[0:1, :] - 2.0 * c0_ref[0:1, :]
         - c1_ref[0:1, :] + c2_ref[0:1, :])
    out_ref[...] = jnp.broadcast_to(
        v.reshape(1, 1, 1, _D, 1), out_ref.shape)


def _broadcast_fill(lat, c0, c1, c2):
    return pl.pallas_call(
        _fill_body,
        grid=(_K // _BLK,),
        in_specs=[
            pl.BlockSpec((1, _D), lambda i: (0, 0)),
            pl.BlockSpec((_K, _D), lambda i: (0, 0)),
            pl.BlockSpec((_K, _D), lambda i: (0, 0)),
            pl.BlockSpec((_K, _D), lambda i: (0, 0)),
        ],
        out_specs=pl.BlockSpec((1, _BLK, _K, _D, _K),
                               lambda i: (0, i, 0, 0, 0)),
        out_shape=jax.ShapeDtypeStruct((1, _K, _K, _D, _K), jnp.float32),
    )(lat, c0, c1, c2)


def kernel(latent_representation, codebook0, codebook1, codebook2):
    out = _broadcast_fill(
        latent_representation, codebook0, codebook1, codebook2)
    return jnp.transpose(out, (0, 1, 2, 4, 3))


# speedup vs baseline: 1.2487x; 1.2487x over previous
import functools

import jax
import jax.numpy as jnp
from jax.experimental import pallas as pl

_K = 64
_D = 32
_BLK = 4


def _fill_body(lat_ref, c0_ref, c1_ref, c2_ref, out_ref):
    v = (3.0 * lat_ref[0:1, :] - 2.0 * c0_ref[0:1, :]
         - c1_ref[0:1, :] + c2_ref[0:1, :])
    out_ref[...] = jnp.broadcast_to(
        v.reshape(1, 1, 1, _D, 1), out_ref.shape)


def _broadcast_fill(lat, c0, c1, c2):
    return pl.pallas_call(
        _fill_body,
        grid=(_K // _BLK,),
        in_specs=[
            pl.BlockSpec((1, _D), lambda i: (0, 0)),
            pl.BlockSpec((_K, _D), lambda i: (0, 0)),
            pl.BlockSpec((_K, _D), lambda i: (0, 0)),
            pl.BlockSpec((_K, _D), lambda i: (0, 0)),
        ],
        out_specs=pl.BlockSpec((1, _BLK, _K, _D, _K),
                               lambda i: (0, i, 0, 0, 0)),
        out_shape=jax.ShapeDtypeStruct((1, _K, _K, _D, _K), jnp.float32),
    )(lat, c0, c1, c2)


def kernel(latent_representation, codebook0, codebook1, codebook2):
    out = _broadcast_fill(
        latent_representation, codebook0, codebook1, codebook2)
    return jnp.transpose(out, (0, 1, 2, 4, 3))


# speedup vs baseline: 1.2688x; 1.0161x over previous
import functools

import jax
import jax.numpy as jnp
from jax.experimental import pallas as pl

_K = 64
_D = 32
_BLK = 4


def _fill_body(lat_ref, c0_ref, c1_ref, c2_ref, out_ref):
    v = (3.0 * lat_ref[0:1, :] - 2.0 * c0_ref[0:1, :]
         - c1_ref[0:1, :] + c2_ref[0:1, :])
    out_ref[...] = jnp.broadcast_to(
        v.reshape(1, 1, 1, _D, 1), out_ref.shape)


def _broadcast_fill(lat, c0, c1, c2):
    return pl.pallas_call(
        _fill_body,
        grid=(_K // _BLK,),
        in_specs=[pl.BlockSpec((1, _D), lambda i: (0, 0))] * 4,
        out_specs=pl.BlockSpec((1, _BLK, _K, _D, _K),
                               lambda i: (0, i, 0, 0, 0)),
        out_shape=jax.ShapeDtypeStruct((1, _K, _K, _D, _K), jnp.float32),
    )(lat, c0, c1, c2)


def kernel(latent_representation, codebook0, codebook1, codebook2):
    out = _broadcast_fill(
        latent_representation, codebook0[0:1], codebook1[0:1], codebook2[0:1])
    return jnp.transpose(out, (0, 1, 2, 4, 3))


# speedup vs baseline: 1.2875x; 1.0147x over previous
import functools

import jax
import jax.numpy as jnp
from jax.experimental import pallas as pl

_K = 64
_D = 32
_BLK = 4


def _fill_body(vecs_ref, out_ref):
    v = (3.0 * vecs_ref[0:1, :] - 2.0 * vecs_ref[1:2, :]
         - vecs_ref[2:3, :] + vecs_ref[3:4, :])
    out_ref[...] = jnp.broadcast_to(
        v.reshape(1, 1, 1, _D, 1), out_ref.shape)


def _broadcast_fill(vecs):
    return pl.pallas_call(
        _fill_body,
        grid=(_K // _BLK,),
        in_specs=[pl.BlockSpec((4, _D), lambda i: (0, 0))],
        out_specs=pl.BlockSpec((1, _BLK, _K, _D, _K),
                               lambda i: (0, i, 0, 0, 0)),
        out_shape=jax.ShapeDtypeStruct((1, _K, _K, _D, _K), jnp.float32),
    )(vecs)


def kernel(latent_representation, codebook0, codebook1, codebook2):
    vecs = jnp.concatenate(
        [latent_representation, codebook0[0:1], codebook1[0:1],
         codebook2[0:1]], axis=0)
    out = _broadcast_fill(vecs)
    return jnp.transpose(out, (0, 1, 2, 4, 3))


# speedup vs baseline: 1.4048x; 1.0912x over previous
import functools

import jax
import jax.numpy as jnp
from jax.experimental import pallas as pl

_K = 64
_D = 32
_BLK = 4


def _fill_body(lat_ref, c0t_ref, c1t_ref, c2t_ref, out_ref):
    lat_t = lat_ref[...].reshape(_D, 1)
    v = (3.0 * lat_t - 2.0 * c0t_ref[:, 0:1]
         - c1t_ref[:, 0:1] + c2t_ref[:, 0:1])
    out_ref[...] = jnp.broadcast_to(
        v.reshape(1, 1, 1, _D, 1), out_ref.shape)


def _broadcast_fill(lat, c0t, c1t, c2t):
    return pl.pallas_call(
        _fill_body,
        grid=(_K // _BLK,),
        in_specs=[
            pl.BlockSpec((1, _D), lambda i: (0, 0)),
            pl.BlockSpec((_D, _K), lambda i: (0, 0)),
            pl.BlockSpec((_D, _K), lambda i: (0, 0)),
            pl.BlockSpec((_D, _K), lambda i: (0, 0)),
        ],
        out_specs=pl.BlockSpec((1, _BLK, _K, _D, _K),
                               lambda i: (0, i, 0, 0, 0)),
        out_shape=jax.ShapeDtypeStruct((1, _K, _K, _D, _K), jnp.float32),
    )(lat, c0t, c1t, c2t)


def kernel(latent_representation, codebook0, codebook1, codebook2):
    out = _broadcast_fill(
        latent_representation, codebook0.T, codebook1.T, codebook2.T)
    return jnp.transpose(out, (0, 1, 2, 4, 3))


# speedup vs baseline: 1.4289x; 1.0171x over previous
import functools

import jax
import jax.numpy as jnp
from jax.experimental import pallas as pl

_K = 64
_D = 32
_BLK = 8


def _fill_body(lat_ref, c0t_ref, c1t_ref, c2t_ref, out_ref):
    lat_t = lat_ref[...].reshape(_D, 1)
    v = (3.0 * lat_t - 2.0 * c0t_ref[:, 0:1]
         - c1t_ref[:, 0:1] + c2t_ref[:, 0:1])
    out_ref[...] = jnp.broadcast_to(
        v.reshape(1, 1, 1, _D, 1), out_ref.shape)


def _broadcast_fill(lat, c0t, c1t, c2t):
    return pl.pallas_call(
        _fill_body,
        grid=(_K // _BLK,),
        in_specs=[
            pl.BlockSpec((1, _D), lambda i: (0, 0)),
            pl.BlockSpec((_D, _K), lambda i: (0, 0)),
            pl.BlockSpec((_D, _K), lambda i: (0, 0)),
            pl.BlockSpec((_D, _K), lambda i: (0, 0)),
        ],
        out_specs=pl.BlockSpec((1, _BLK, _K, _D, _K),
                               lambda i: (0, i, 0, 0, 0)),
        out_shape=jax.ShapeDtypeStruct((1, _K, _K, _D, _K), jnp.float32),
    )(lat, c0t, c1t, c2t)


def kernel(latent_representation, codebook0, codebook1, codebook2):
    out = _broadcast_fill(
        latent_representation, codebook0.T, codebook1.T, codebook2.T)
    return jnp.transpose(out, (0, 1, 2, 4, 3))
